# baseline (speedup 1.0000x reference)
import functools

import jax
import jax.numpy as jnp
from jax import lax
from jax.experimental import pallas as pl
from jax.experimental.pallas import tpu as pltpu
from jax.experimental.pallas import tpu_sc as plsc

N = 10000
E = 320000
D = 128
EPS = 1e-5

NC = 2
NS = 16
NT = NC * NS
K = 128
NPAD = 10240
EPAD = 327680
EPT = EPAD // NT
NCH = EPT // K
RPT = NPAD // NS


@functools.cache
def _build_deg():
    mesh = plsc.VectorSubcoreMesh(core_axis_name="c", subcore_axis_name="s",
                                  num_cores=NC, num_subcores=NS)
    return functools.partial(
        pl.kernel,
        out_type=jax.ShapeDtypeStruct((NC, NPAD, D), jnp.float32),
        mesh=mesh,
        scratch_types=[
            pltpu.VMEM((2 * NCH, K), jnp.int32),
            pltpu.VMEM((K, D), jnp.float32),
            pltpu.VMEM_SHARED((NPAD, D), jnp.float32),
            pltpu.SemaphoreType.DMA,
        ],
    )(_deg_body)


def _deg_body(il_hbm, e01_hbm, zeros_hbm, out_hbm, il_v, e01_v, acc_sh, sem):
    c = lax.axis_index("c")
    s = lax.axis_index("s")
    wid = s * NC + c
    pltpu.sync_copy(il_hbm.at[wid], il_v)
    pltpu.sync_copy(e01_hbm, e01_v)
    pltpu.sync_copy(zeros_hbm.at[pl.ds(s * RPT, RPT)],
                    acc_sh.at[pl.ds(s * RPT, RPT)])
    plsc.subcore_barrier()

    def body(j, carry):
        pltpu.async_copy(e01_v, acc_sh.at[il_v.at[j]], sem, add=True)

        @pl.when(j >= 2)
        def _():
            pltpu.make_async_copy(e01_v, acc_sh.at[il_v.at[j]], sem).wait()

        return carry

    lax.fori_loop(0, 2 * NCH, body, 0)
    pltpu.make_async_copy(e01_v, acc_sh.at[il_v.at[0]], sem).wait()
    pltpu.make_async_copy(e01_v, acc_sh.at[il_v.at[0]], sem).wait()
    plsc.subcore_barrier()
    pltpu.sync_copy(acc_sh.at[pl.ds(s * RPT, RPT)],
                    out_hbm.at[c, pl.ds(s * RPT, RPT)])


FAST_CORE = 0
GC = 4
NG = (2 * NCH) // GC
FG = 32
SG = NG - FG


@functools.cache
def _build_agg():
    mesh = plsc.VectorSubcoreMesh(core_axis_name="c", subcore_axis_name="s",
                                  num_cores=NC, num_subcores=NS)
    return functools.partial(
        pl.kernel,
        out_type=jax.ShapeDtypeStruct((NC, NPAD, D), jnp.float32),
        mesh=mesh,
        scratch_types=[
            pltpu.VMEM((2, 2 * GC, K), jnp.int32),
            pltpu.VMEM((K, D), jnp.float32),
            pltpu.VMEM((K, D), jnp.float32),
            pltpu.VMEM_SHARED((NPAD, D), jnp.float32),
            pltpu.SemaphoreType.DMA,
            pltpu.SemaphoreType.DMA,
            pltpu.SemaphoreType.DMA,
        ],
    )(_agg_body)


def _agg_body(hp_hbm, ilg_hbm, zeros_hbm, out_hbm,
              il_v, rb0, rb1, acc_sh, sem_g, sem_s, sem_i):
    c = lax.axis_index("c")
    s = lax.axis_index("s")
    fast = c == FAST_CORE
    gbase = jnp.where(fast, 0, FG)
    ng = jnp.where(fast, FG, SG)
    rbufs = (rb0, rb1)

    pltpu.sync_copy(zeros_hbm.at[pl.ds(s * RPT, RPT)],
                    acc_sh.at[pl.ds(s * RPT, RPT)])
    pltpu.sync_copy(ilg_hbm.at[s, gbase], il_v.at[0])
    plsc.subcore_barrier()
    pltpu.async_copy(hp_hbm.at[il_v.at[0, 0]], rb0, sem_g)

    def wait_gather(buf):
        pltpu.make_async_copy(hp_hbm.at[il_v.at[0, 0]], buf, sem_g).wait()

    def drain_scatter(buf):
        pltpu.make_async_copy(buf, acc_sh.at[il_v.at[0, 1]], sem_s).wait()

    def body(t, carry):
        slot = t % 2
        nslot = (t + 1) % 2

        @pl.when(t + 1 < ng)
        def _():
            pltpu.async_copy(ilg_hbm.at[s, gbase + t + 1], il_v.at[nslot],
                             sem_i)

        for i in range(GC):
            rb_cur = rbufs[i % 2]
            rb_nxt = rbufs[(i + 1) % 2]
            wait_gather(rb_cur)
            pltpu.async_copy(rb_cur, acc_sh.at[il_v.at[slot, 2 * i + 1]],
                             sem_s, add=True)
            if i == 0:
                @pl.when(t > 0)
                def _():
                    drain_scatter(rb_nxt)
            else:
                drain_scatter(rb_nxt)
            if i < GC - 1:
                pltpu.async_copy(hp_hbm.at[il_v.at[slot, 2 * (i + 1)]],
                                 rb_nxt, sem_g)
            else:
                @pl.when(t + 1 < ng)
                def _():
                    pltpu.make_async_copy(ilg_hbm.at[s, gbase],
                                          il_v.at[nslot], sem_i).wait()
                    pltpu.async_copy(hp_hbm.at[il_v.at[nslot, 0]], rb_nxt,
                                     sem_g)
        return carry

    lax.fori_loop(0, ng, body, 0)
    drain_scatter(rbufs[(GC - 1) % 2])
    plsc.subcore_barrier()
    pltpu.sync_copy(acc_sh.at[pl.ds(s * RPT, RPT)],
                    out_hbm.at[c, pl.ds(s * RPT, RPT)])


BM = 512
BML = 1000


def _norms(degq, col, row0, nrows):
    deg = degq[:, col:col + 1] + degq[:, col + 2:col + 3]
    rows = row0 + lax.broadcasted_iota(jnp.int32, (nrows, 1), 0)
    ok = (deg > 0) & (rows < N)
    return jnp.where(ok, lax.rsqrt(jnp.maximum(deg, 1.0)), 0.0)


def _pre_body(x_ref, w_ref, dq_ref, o_ref):
    m = pl.program_id(0)
    nsrc = _norms(dq_ref[...], 0, m * BM, BM)
    h = jnp.dot(x_ref[...], w_ref[...], preferred_element_type=jnp.float32)
    o_ref[...] = h * nsrc


def _tc_pre(x_pad, W, degq):
    return pl.pallas_call(
        _pre_body,
        grid=(NPAD // BM,),
        in_specs=[
            pl.BlockSpec((BM, D), lambda m: (m, 0)),
            pl.BlockSpec((D, D), lambda m: (0, 0)),
            pl.BlockSpec((BM, 4), lambda m: (m, 0)),
        ],
        out_specs=pl.BlockSpec((BM, D), lambda m: (m, 0)),
        out_shape=jax.ShapeDtypeStruct((NPAD, D), jnp.float32),
    )(x_pad, W, degq)


def _layer_h(p_ref, dq, b_ref, g_ref, be_ref, row0, nrows):
    ndst = _norms(dq, 1, row0, nrows)
    agg = (p_ref[0] + p_ref[1]) * ndst
    a1 = jnp.maximum(agg + b_ref[...], 0.0)
    gs = g_ref[...] * lax.rsqrt(jnp.float32(1.0 + EPS))
    return jnp.maximum(a1 * gs + be_ref[...], 0.0)


def _mid_body(p_ref, dq_ref, b_ref, g_ref, be_ref, w_ref,
              h_ref, hp_ref):
    m = pl.program_id(0)
    dq = dq_ref[...]
    h = _layer_h(p_ref, dq, b_ref, g_ref, be_ref, m * BM, BM)
    h_ref[...] = h
    nsrc = _norms(dq, 0, m * BM, BM)
    hp_ref[...] = jnp.dot(h, w_ref[...],
                          preferred_element_type=jnp.float32) * nsrc


def _tc_mid(partials, degq, b, g, be, Wnext):
    return pl.pallas_call(
        _mid_body,
        grid=(NPAD // BM,),
        in_specs=[
            pl.BlockSpec((NC, BM, D), lambda m: (0, m, 0)),
            pl.BlockSpec((BM, 4), lambda m: (m, 0)),
            pl.BlockSpec((1, D), lambda m: (0, 0)),
            pl.BlockSpec((1, D), lambda m: (0, 0)),
            pl.BlockSpec((1, D), lambda m: (0, 0)),
            pl.BlockSpec((D, D), lambda m: (0, 0)),
        ],
        out_specs=[
            pl.BlockSpec((BM, D), lambda m: (m, 0)),
            pl.BlockSpec((BM, D), lambda m: (m, 0)),
        ],
        out_shape=[
            jax.ShapeDtypeStruct((NPAD, D), jnp.float32),
            jax.ShapeDtypeStruct((NPAD, D), jnp.float32),
        ],
    )(partials, degq, b, g, be, Wnext)


def _last_body(p_ref, dq_ref, b_ref, g_ref, be_ref, h0_ref, h1_ref,
               lw_ref, lb_ref, o_ref):
    m = pl.program_id(0)
    h2 = _layer_h(p_ref, dq_ref[...], b_ref, g_ref, be_ref, m * BML, BML)
    jk = jnp.maximum(jnp.maximum(h0_ref[...], h1_ref[...]), h2)
    o_ref[...] = jnp.maximum(
        jnp.dot(jk, lw_ref[...], preferred_element_type=jnp.float32)
        + lb_ref[...], 0.0)


def _tc_last(partials, degq, b, g, be, h0, h1, lin1_W, lin1_b):
    return pl.pallas_call(
        _last_body,
        grid=(N // BML,),
        in_specs=[
            pl.BlockSpec((NC, BML, D), lambda m: (0, m, 0)),
            pl.BlockSpec((BML, 4), lambda m: (m, 0)),
            pl.BlockSpec((1, D), lambda m: (0, 0)),
            pl.BlockSpec((1, D), lambda m: (0, 0)),
            pl.BlockSpec((1, D), lambda m: (0, 0)),
            pl.BlockSpec((BML, D), lambda m: (m, 0)),
            pl.BlockSpec((BML, D), lambda m: (m, 0)),
            pl.BlockSpec((D, D), lambda m: (0, 0)),
            pl.BlockSpec((1, D), lambda m: (0, 0)),
        ],
        out_specs=pl.BlockSpec((BML, D), lambda m: (m, 0)),
        out_shape=jax.ShapeDtypeStruct((N, D), jnp.float32),
    )(partials, degq, b, g, be, h0, h1, lin1_W, lin1_b)


def kernel(adj_t, x, W0, b0, g0, be0, W1, b1, g1, be1, W2, b2, g2, be2,
           lin1_W, lin1_b):
    pad = EPAD - E
    padv = jnp.full((pad,), N, jnp.int32)
    srcp = jnp.concatenate([adj_t[0], padv])
    dstp = jnp.concatenate([adj_t[1], padv])
    srcr = srcp.reshape(NS, NG, GC, K)
    dstr = dstp.reshape(NS, NG, GC, K)
    ilg = jnp.stack([srcr, dstr], axis=3).reshape(NS, NG, 2 * GC, K)
    x_pad = jnp.pad(x, ((0, NPAD - N), (0, 0)))
    zeros = jnp.zeros((NPAD, D), jnp.float32)
    il3 = jnp.stack([srcp.reshape(NT, EPT), dstp.reshape(NT, EPT)],
                    axis=-1).reshape(NT, 2 * NCH, K)
    eye2 = jnp.eye(2, D, dtype=jnp.float32)
    e01 = jnp.tile(eye2, (K // 2, 1))

    degp = _build_deg()(il3, e01, zeros)
    degq = degp[:, :, :2].transpose(1, 0, 2).reshape(NPAD, NC * 2)

    b = [b0.reshape(1, D), b1.reshape(1, D), b2.reshape(1, D)]
    g = [g0.reshape(1, D), g1.reshape(1, D), g2.reshape(1, D)]
    be = [be0.reshape(1, D), be1.reshape(1, D), be2.reshape(1, D)]

    hp = _tc_pre(x_pad, W0, degq)
    p0 = _build_agg()(hp, ilg, zeros)
    h0, hp = _tc_mid(p0, degq, b[0], g[0], be[0], W1)
    p1 = _build_agg()(hp, ilg, zeros)
    h1, hp = _tc_mid(p1, degq, b[1], g[1], be[1], W2)
    p2 = _build_agg()(hp, ilg, zeros)
    out = _tc_last(p2, degq, b[2], g[2], be[2], h0, h1,
                   lin1_W, lin1_b.reshape(1, D))
    return out


# speedup vs baseline: 1.0422x; 1.0422x over previous
import functools

import jax
import jax.numpy as jnp
from jax import lax
from jax.experimental import pallas as pl
from jax.experimental.pallas import tpu as pltpu
from jax.experimental.pallas import tpu_sc as plsc

N = 10000
E = 320000
D = 128
EPS = 1e-5

NC = 2
NS = 16
NT = NC * NS
K = 128
NPAD = 10240
EPAD = 327680
EPT = EPAD // NT
NCH = EPT // K
RPT = NPAD // NS


@functools.cache
def _build_deg():
    mesh = plsc.VectorSubcoreMesh(core_axis_name="c", subcore_axis_name="s",
                                  num_cores=NC, num_subcores=NS)
    return functools.partial(
        pl.kernel,
        out_type=jax.ShapeDtypeStruct((NC, NPAD, D), jnp.float32),
        mesh=mesh,
        scratch_types=[
            pltpu.VMEM((2 * NCH, K), jnp.int32),
            pltpu.VMEM((K, D), jnp.float32),
            pltpu.VMEM_SHARED((NPAD, D), jnp.float32),
            pltpu.SemaphoreType.DMA,
        ],
    )(_deg_body)


def _deg_body(il_hbm, e01_hbm, zeros_hbm, out_hbm, il_v, e01_v, acc_sh, sem):
    c = lax.axis_index("c")
    s = lax.axis_index("s")
    wid = s * NC + c
    pltpu.sync_copy(il_hbm.at[wid], il_v)
    pltpu.sync_copy(e01_hbm, e01_v)
    pltpu.sync_copy(zeros_hbm.at[pl.ds(s * RPT, RPT)],
                    acc_sh.at[pl.ds(s * RPT, RPT)])
    plsc.subcore_barrier()

    def body(j, carry):
        pltpu.async_copy(e01_v, acc_sh.at[il_v.at[j]], sem, add=True)

        @pl.when(j >= 2)
        def _():
            pltpu.make_async_copy(e01_v, acc_sh.at[il_v.at[j]], sem).wait()

        return carry

    lax.fori_loop(0, 2 * NCH, body, 0)
    pltpu.make_async_copy(e01_v, acc_sh.at[il_v.at[0]], sem).wait()
    pltpu.make_async_copy(e01_v, acc_sh.at[il_v.at[0]], sem).wait()
    plsc.subcore_barrier()
    pltpu.sync_copy(acc_sh.at[pl.ds(s * RPT, RPT)],
                    out_hbm.at[c, pl.ds(s * RPT, RPT)])


FAST_CORE = 0
KA = 80
GC = 4
NG = EPT * 2 // (GC * KA)
FG = 48
SG = NG - FG
NB = 3


@functools.cache
def _build_agg():
    mesh = plsc.VectorSubcoreMesh(core_axis_name="c", subcore_axis_name="s",
                                  num_cores=NC, num_subcores=NS)
    return functools.partial(
        pl.kernel,
        out_type=jax.ShapeDtypeStruct((NC, NPAD, D), jnp.float32),
        mesh=mesh,
        scratch_types=[
            pltpu.VMEM((2, 2 * GC, KA), jnp.int32),
            pltpu.VMEM((NB, KA, D), jnp.float32),
            pltpu.VMEM_SHARED((NPAD, D), jnp.float32),
            pltpu.SemaphoreType.DMA,
            pltpu.SemaphoreType.DMA,
            pltpu.SemaphoreType.DMA,
        ],
    )(_agg_body)


def _agg_body(hp_hbm, ilg_hbm, zeros_hbm, out_hbm,
              il_v, rb_v, acc_sh, sem_g, sem_s, sem_i):
    c = lax.axis_index("c")
    s = lax.axis_index("s")
    fast = c == FAST_CORE
    gbase = jnp.where(fast, 0, FG)
    ng = jnp.where(fast, FG, SG)

    pltpu.sync_copy(zeros_hbm.at[pl.ds(s * RPT, RPT)],
                    acc_sh.at[pl.ds(s * RPT, RPT)])
    pltpu.sync_copy(ilg_hbm.at[s, gbase], il_v.at[0])
    plsc.subcore_barrier()
    pltpu.async_copy(hp_hbm.at[il_v.at[0, 0]], rb_v.at[0], sem_g)
    pltpu.async_copy(hp_hbm.at[il_v.at[0, 2]], rb_v.at[1], sem_g)

    def wait_gather():
        pltpu.make_async_copy(hp_hbm.at[il_v.at[0, 0]], rb_v.at[0],
                              sem_g).wait()

    def drain_scatter():
        pltpu.make_async_copy(rb_v.at[0], acc_sh.at[il_v.at[0, 1]],
                              sem_s).wait()

    def body(t, carry):
        slot = t % 2
        nslot = (t + 1) % 2

        @pl.when(t + 1 < ng)
        def _():
            pltpu.async_copy(ilg_hbm.at[s, gbase + t + 1], il_v.at[nslot],
                             sem_i)

        j0 = GC * t
        for i in range(GC):
            bm = (j0 + i) % NB
            wait_gather()
            pltpu.async_copy(rb_v.at[bm], acc_sh.at[il_v.at[slot, 2 * i + 1]],
                             sem_s, add=True)
            if i == 0:
                @pl.when(t > 0)
                def _():
                    drain_scatter()
            else:
                drain_scatter()
            bn = (j0 + i + 2) % NB
            if i + 2 < GC:
                pltpu.async_copy(hp_hbm.at[il_v.at[slot, 2 * (i + 2)]],
                                 rb_v.at[bn], sem_g)
            elif i + 2 == GC:
                @pl.when(t + 1 < ng)
                def _():
                    pltpu.make_async_copy(ilg_hbm.at[s, gbase],
                                          il_v.at[nslot], sem_i).wait()
                    pltpu.async_copy(hp_hbm.at[il_v.at[nslot, 0]],
                                     rb_v.at[bn], sem_g)
            else:
                @pl.when(t + 1 < ng)
                def _():
                    pltpu.async_copy(hp_hbm.at[il_v.at[nslot, 2]],
                                     rb_v.at[bn], sem_g)
        return carry

    lax.fori_loop(0, ng, body, 0)
    drain_scatter()
    plsc.subcore_barrier()
    pltpu.sync_copy(acc_sh.at[pl.ds(s * RPT, RPT)],
                    out_hbm.at[c, pl.ds(s * RPT, RPT)])


BM = 512
BML = 1000


def _norms(degq, col, row0, nrows):
    deg = degq[:, col:col + 1] + degq[:, col + 2:col + 3]
    rows = row0 + lax.broadcasted_iota(jnp.int32, (nrows, 1), 0)
    ok = (deg > 0) & (rows < N)
    return jnp.where(ok, lax.rsqrt(jnp.maximum(deg, 1.0)), 0.0)


def _pre_body(x_ref, w_ref, dq_ref, o_ref):
    m = pl.program_id(0)
    nsrc = _norms(dq_ref[...], 0, m * BM, BM)
    h = jnp.dot(x_ref[...], w_ref[...], preferred_element_type=jnp.float32)
    o_ref[...] = h * nsrc


def _tc_pre(x_pad, W, degq):
    return pl.pallas_call(
        _pre_body,
        grid=(NPAD // BM,),
        in_specs=[
            pl.BlockSpec((BM, D), lambda m: (m, 0)),
            pl.BlockSpec((D, D), lambda m: (0, 0)),
            pl.BlockSpec((BM, 4), lambda m: (m, 0)),
        ],
        out_specs=pl.BlockSpec((BM, D), lambda m: (m, 0)),
        out_shape=jax.ShapeDtypeStruct((NPAD, D), jnp.float32),
    )(x_pad, W, degq)


def _layer_h(p_ref, dq, b_ref, g_ref, be_ref, row0, nrows):
    ndst = _norms(dq, 1, row0, nrows)
    agg = (p_ref[0] + p_ref[1]) * ndst
    a1 = jnp.maximum(agg + b_ref[...], 0.0)
    gs = g_ref[...] * lax.rsqrt(jnp.float32(1.0 + EPS))
    return jnp.maximum(a1 * gs + be_ref[...], 0.0)


def _mid_body(p_ref, dq_ref, b_ref, g_ref, be_ref, w_ref,
              h_ref, hp_ref):
    m = pl.program_id(0)
    dq = dq_ref[...]
    h = _layer_h(p_ref, dq, b_ref, g_ref, be_ref, m * BM, BM)
    h_ref[...] = h
    nsrc = _norms(dq, 0, m * BM, BM)
    hp_ref[...] = jnp.dot(h, w_ref[...],
                          preferred_element_type=jnp.float32) * nsrc


def _tc_mid(partials, degq, b, g, be, Wnext):
    return pl.pallas_call(
        _mid_body,
        grid=(NPAD // BM,),
        in_specs=[
            pl.BlockSpec((NC, BM, D), lambda m: (0, m, 0)),
            pl.BlockSpec((BM, 4), lambda m: (m, 0)),
            pl.BlockSpec((1, D), lambda m: (0, 0)),
            pl.BlockSpec((1, D), lambda m: (0, 0)),
            pl.BlockSpec((1, D), lambda m: (0, 0)),
            pl.BlockSpec((D, D), lambda m: (0, 0)),
        ],
        out_specs=[
            pl.BlockSpec((BM, D), lambda m: (m, 0)),
            pl.BlockSpec((BM, D), lambda m: (m, 0)),
        ],
        out_shape=[
            jax.ShapeDtypeStruct((NPAD, D), jnp.float32),
            jax.ShapeDtypeStruct((NPAD, D), jnp.float32),
        ],
    )(partials, degq, b, g, be, Wnext)


def _last_body(p_ref, dq_ref, b_ref, g_ref, be_ref, h0_ref, h1_ref,
               lw_ref, lb_ref, o_ref):
    m = pl.program_id(0)
    h2 = _layer_h(p_ref, dq_ref[...], b_ref, g_ref, be_ref, m * BML, BML)
    jk = jnp.maximum(jnp.maximum(h0_ref[...], h1_ref[...]), h2)
    o_ref[...] = jnp.maximum(
        jnp.dot(jk, lw_ref[...], preferred_element_type=jnp.float32)
        + lb_ref[...], 0.0)


def _tc_last(partials, degq, b, g, be, h0, h1, lin1_W, lin1_b):
    return pl.pallas_call(
        _last_body,
        grid=(N // BML,),
        in_specs=[
            pl.BlockSpec((NC, BML, D), lambda m: (0, m, 0)),
            pl.BlockSpec((BML, 4), lambda m: (m, 0)),
            pl.BlockSpec((1, D), lambda m: (0, 0)),
            pl.BlockSpec((1, D), lambda m: (0, 0)),
            pl.BlockSpec((1, D), lambda m: (0, 0)),
            pl.BlockSpec((BML, D), lambda m: (m, 0)),
            pl.BlockSpec((BML, D), lambda m: (m, 0)),
            pl.BlockSpec((D, D), lambda m: (0, 0)),
            pl.BlockSpec((1, D), lambda m: (0, 0)),
        ],
        out_specs=pl.BlockSpec((BML, D), lambda m: (m, 0)),
        out_shape=jax.ShapeDtypeStruct((N, D), jnp.float32),
    )(partials, degq, b, g, be, h0, h1, lin1_W, lin1_b)


def kernel(adj_t, x, W0, b0, g0, be0, W1, b1, g1, be1, W2, b2, g2, be2,
           lin1_W, lin1_b):
    pad = EPAD - E
    padv = jnp.full((pad,), N, jnp.int32)
    srcp = jnp.concatenate([adj_t[0], padv])
    dstp = jnp.concatenate([adj_t[1], padv])
    srcr = srcp.reshape(NS, NG, GC, KA)
    dstr = dstp.reshape(NS, NG, GC, KA)
    ilg = jnp.stack([srcr, dstr], axis=3).reshape(NS, NG, 2 * GC, KA)
    x_pad = jnp.pad(x, ((0, NPAD - N), (0, 0)))
    zeros = jnp.zeros((NPAD, D), jnp.float32)
    il3 = jnp.stack([srcp.reshape(NT, EPT), dstp.reshape(NT, EPT)],
                    axis=-1).reshape(NT, 2 * NCH, K)
    eye2 = jnp.eye(2, D, dtype=jnp.float32)
    e01 = jnp.tile(eye2, (K // 2, 1))

    degp = _build_deg()(il3, e01, zeros)
    degq = degp[:, :, :2].transpose(1, 0, 2).reshape(NPAD, NC * 2)

    b = [b0.reshape(1, D), b1.reshape(1, D), b2.reshape(1, D)]
    g = [g0.reshape(1, D), g1.reshape(1, D), g2.reshape(1, D)]
    be = [be0.reshape(1, D), be1.reshape(1, D), be2.reshape(1, D)]

    hp = _tc_pre(x_pad, W0, degq)
    p0 = _build_agg()(hp, ilg, zeros)
    h0, hp = _tc_mid(p0, degq, b[0], g[0], be[0], W1)
    p1 = _build_agg()(hp, ilg, zeros)
    h1, hp = _tc_mid(p1, degq, b[1], g[1], be[1], W2)
    p2 = _build_agg()(hp, ilg, zeros)
    out = _tc_last(p2, degq, b[2], g[2], be[2], h0, h1,
                   lin1_W, lin1_b.reshape(1, D))
    return out


# speedup vs baseline: 1.0531x; 1.0105x over previous
import functools

import jax
import jax.numpy as jnp
from jax import lax
from jax.experimental import pallas as pl
from jax.experimental.pallas import tpu as pltpu
from jax.experimental.pallas import tpu_sc as plsc

N = 10000
E = 320000
D = 128
EPS = 1e-5

NC = 2
NS = 16
NT = NC * NS
K = 128
NPAD = 10240
EPAD = 327680
EPT = EPAD // NT
NCH = EPT // K
RPT = NPAD // NS


@functools.cache
def _build_deg():
    mesh = plsc.VectorSubcoreMesh(core_axis_name="c", subcore_axis_name="s",
                                  num_cores=NC, num_subcores=NS)
    return functools.partial(
        pl.kernel,
        out_type=jax.ShapeDtypeStruct((NC, NPAD, D), jnp.float32),
        mesh=mesh,
        scratch_types=[
            pltpu.VMEM((2 * NCH, K), jnp.int32),
            pltpu.VMEM((K, D), jnp.float32),
            pltpu.VMEM_SHARED((NPAD, D), jnp.float32),
            pltpu.SemaphoreType.DMA,
        ],
    )(_deg_body)


def _deg_body(il_hbm, e01_hbm, zeros_hbm, out_hbm, il_v, e01_v, acc_sh, sem):
    c = lax.axis_index("c")
    s = lax.axis_index("s")
    wid = s * NC + c
    pltpu.sync_copy(il_hbm.at[wid], il_v)
    pltpu.sync_copy(e01_hbm, e01_v)
    pltpu.sync_copy(zeros_hbm.at[pl.ds(s * RPT, RPT)],
                    acc_sh.at[pl.ds(s * RPT, RPT)])
    plsc.subcore_barrier()

    def body(j, carry):
        pltpu.async_copy(e01_v, acc_sh.at[il_v.at[j]], sem, add=True)

        @pl.when(j >= 2)
        def _():
            pltpu.make_async_copy(e01_v, acc_sh.at[il_v.at[j]], sem).wait()

        return carry

    lax.fori_loop(0, 2 * NCH, body, 0)
    pltpu.make_async_copy(e01_v, acc_sh.at[il_v.at[0]], sem).wait()
    pltpu.make_async_copy(e01_v, acc_sh.at[il_v.at[0]], sem).wait()
    plsc.subcore_barrier()
    pltpu.sync_copy(acc_sh.at[pl.ds(s * RPT, RPT)],
                    out_hbm.at[c, pl.ds(s * RPT, RPT)])


FAST_CORE = 0
KA = 80
GC = 4
NG = EPT * 2 // (GC * KA)
FG = 50
SG = NG - FG
NB = 3


@functools.cache
def _build_agg():
    mesh = plsc.VectorSubcoreMesh(core_axis_name="c", subcore_axis_name="s",
                                  num_cores=NC, num_subcores=NS)
    return functools.partial(
        pl.kernel,
        out_type=jax.ShapeDtypeStruct((NC, NPAD, D), jnp.float32),
        mesh=mesh,
        scratch_types=[
            pltpu.VMEM((2, 2 * GC, KA), jnp.int32),
            pltpu.VMEM((NB, KA, D), jnp.float32),
            pltpu.VMEM_SHARED((NPAD, D), jnp.float32),
            pltpu.SemaphoreType.DMA,
            pltpu.SemaphoreType.DMA,
            pltpu.SemaphoreType.DMA,
        ],
    )(_agg_body)


def _agg_body(hp_hbm, ilg_hbm, zeros_hbm, out_hbm,
              il_v, rb_v, acc_sh, sem_g, sem_s, sem_i):
    c = lax.axis_index("c")
    s = lax.axis_index("s")
    fast = c == FAST_CORE
    gbase = jnp.where(fast, 0, FG)
    ng = jnp.where(fast, FG, SG)

    pltpu.sync_copy(zeros_hbm.at[pl.ds(s * RPT, RPT)],
                    acc_sh.at[pl.ds(s * RPT, RPT)])
    pltpu.sync_copy(ilg_hbm.at[s, gbase], il_v.at[0])
    plsc.subcore_barrier()
    pltpu.async_copy(hp_hbm.at[il_v.at[0, 0]], rb_v.at[0], sem_g)
    pltpu.async_copy(hp_hbm.at[il_v.at[0, 2]], rb_v.at[1], sem_g)

    def wait_gather():
        pltpu.make_async_copy(hp_hbm.at[il_v.at[0, 0]], rb_v.at[0],
                              sem_g).wait()

    def drain_scatter():
        pltpu.make_async_copy(rb_v.at[0], acc_sh.at[il_v.at[0, 1]],
                              sem_s).wait()

    def body(t, carry):
        slot = t % 2
        nslot = (t + 1) % 2

        @pl.when(t + 1 < ng)
        def _():
            pltpu.async_copy(ilg_hbm.at[s, gbase + t + 1], il_v.at[nslot],
                             sem_i)

        j0 = GC * t
        for i in range(GC):
            bm = (j0 + i) % NB
            wait_gather()
            pltpu.async_copy(rb_v.at[bm], acc_sh.at[il_v.at[slot, 2 * i + 1]],
                             sem_s, add=True)
            if i == 0:
                @pl.when(t > 0)
                def _():
                    drain_scatter()
            else:
                drain_scatter()
            bn = (j0 + i + 2) % NB
            if i + 2 < GC:
                pltpu.async_copy(hp_hbm.at[il_v.at[slot, 2 * (i + 2)]],
                                 rb_v.at[bn], sem_g)
            elif i + 2 == GC:
                @pl.when(t + 1 < ng)
                def _():
                    pltpu.make_async_copy(ilg_hbm.at[s, gbase],
                                          il_v.at[nslot], sem_i).wait()
                    pltpu.async_copy(hp_hbm.at[il_v.at[nslot, 0]],
                                     rb_v.at[bn], sem_g)
            else:
                @pl.when(t + 1 < ng)
                def _():
                    pltpu.async_copy(hp_hbm.at[il_v.at[nslot, 2]],
                                     rb_v.at[bn], sem_g)
        return carry

    lax.fori_loop(0, ng, body, 0)
    drain_scatter()
    plsc.subcore_barrier()
    pltpu.sync_copy(acc_sh.at[pl.ds(s * RPT, RPT)],
                    out_hbm.at[c, pl.ds(s * RPT, RPT)])


BM = 512
BML = 1000


def _norms(degq, col, row0, nrows):
    deg = degq[:, col:col + 1] + degq[:, col + 2:col + 3]
    rows = row0 + lax.broadcasted_iota(jnp.int32, (nrows, 1), 0)
    ok = (deg > 0) & (rows < N)
    return jnp.where(ok, lax.rsqrt(jnp.maximum(deg, 1.0)), 0.0)


def _pre_body(x_ref, w_ref, dq_ref, o_ref):
    m = pl.program_id(0)
    nsrc = _norms(dq_ref[...], 0, m * BM, BM)
    h = jnp.dot(x_ref[...], w_ref[...], preferred_element_type=jnp.float32)
    o_ref[...] = h * nsrc


def _tc_pre(x_pad, W, degq):
    return pl.pallas_call(
        _pre_body,
        grid=(NPAD // BM,),
        in_specs=[
            pl.BlockSpec((BM, D), lambda m: (m, 0)),
            pl.BlockSpec((D, D), lambda m: (0, 0)),
            pl.BlockSpec((BM, 4), lambda m: (m, 0)),
        ],
        out_specs=pl.BlockSpec((BM, D), lambda m: (m, 0)),
        out_shape=jax.ShapeDtypeStruct((NPAD, D), jnp.float32),
    )(x_pad, W, degq)


def _layer_h(p_ref, dq, b_ref, g_ref, be_ref, row0, nrows):
    ndst = _norms(dq, 1, row0, nrows)
    agg = (p_ref[0] + p_ref[1]) * ndst
    a1 = jnp.maximum(agg + b_ref[...], 0.0)
    gs = g_ref[...] * lax.rsqrt(jnp.float32(1.0 + EPS))
    return jnp.maximum(a1 * gs + be_ref[...], 0.0)


def _mid_body(p_ref, dq_ref, b_ref, g_ref, be_ref, w_ref,
              h_ref, hp_ref):
    m = pl.program_id(0)
    dq = dq_ref[...]
    h = _layer_h(p_ref, dq, b_ref, g_ref, be_ref, m * BM, BM)
    h_ref[...] = h
    nsrc = _norms(dq, 0, m * BM, BM)
    hp_ref[...] = jnp.dot(h, w_ref[...],
                          preferred_element_type=jnp.float32) * nsrc


def _tc_mid(partials, degq, b, g, be, Wnext):
    return pl.pallas_call(
        _mid_body,
        grid=(NPAD // BM,),
        in_specs=[
            pl.BlockSpec((NC, BM, D), lambda m: (0, m, 0)),
            pl.BlockSpec((BM, 4), lambda m: (m, 0)),
            pl.BlockSpec((1, D), lambda m: (0, 0)),
            pl.BlockSpec((1, D), lambda m: (0, 0)),
            pl.BlockSpec((1, D), lambda m: (0, 0)),
            pl.BlockSpec((D, D), lambda m: (0, 0)),
        ],
        out_specs=[
            pl.BlockSpec((BM, D), lambda m: (m, 0)),
            pl.BlockSpec((BM, D), lambda m: (m, 0)),
        ],
        out_shape=[
            jax.ShapeDtypeStruct((NPAD, D), jnp.float32),
            jax.ShapeDtypeStruct((NPAD, D), jnp.float32),
        ],
    )(partials, degq, b, g, be, Wnext)


def _last_body(p_ref, dq_ref, b_ref, g_ref, be_ref, h0_ref, h1_ref,
               lw_ref, lb_ref, o_ref):
    m = pl.program_id(0)
    h2 = _layer_h(p_ref, dq_ref[...], b_ref, g_ref, be_ref, m * BML, BML)
    jk = jnp.maximum(jnp.maximum(h0_ref[...], h1_ref[...]), h2)
    o_ref[...] = jnp.maximum(
        jnp.dot(jk, lw_ref[...], preferred_element_type=jnp.float32)
        + lb_ref[...], 0.0)


def _tc_last(partials, degq, b, g, be, h0, h1, lin1_W, lin1_b):
    return pl.pallas_call(
        _last_body,
        grid=(N // BML,),
        in_specs=[
            pl.BlockSpec((NC, BML, D), lambda m: (0, m, 0)),
            pl.BlockSpec((BML, 4), lambda m: (m, 0)),
            pl.BlockSpec((1, D), lambda m: (0, 0)),
            pl.BlockSpec((1, D), lambda m: (0, 0)),
            pl.BlockSpec((1, D), lambda m: (0, 0)),
            pl.BlockSpec((BML, D), lambda m: (m, 0)),
            pl.BlockSpec((BML, D), lambda m: (m, 0)),
            pl.BlockSpec((D, D), lambda m: (0, 0)),
            pl.BlockSpec((1, D), lambda m: (0, 0)),
        ],
        out_specs=pl.BlockSpec((BML, D), lambda m: (m, 0)),
        out_shape=jax.ShapeDtypeStruct((N, D), jnp.float32),
    )(partials, degq, b, g, be, h0, h1, lin1_W, lin1_b)


def kernel(adj_t, x, W0, b0, g0, be0, W1, b1, g1, be1, W2, b2, g2, be2,
           lin1_W, lin1_b):
    pad = EPAD - E
    padv = jnp.full((pad,), N, jnp.int32)
    srcp = jnp.concatenate([adj_t[0], padv])
    dstp = jnp.concatenate([adj_t[1], padv])
    srcr = srcp.reshape(NS, NG, GC, KA)
    dstr = dstp.reshape(NS, NG, GC, KA)
    ilg = jnp.stack([srcr, dstr], axis=3).reshape(NS, NG, 2 * GC, KA)
    x_pad = jnp.pad(x, ((0, NPAD - N), (0, 0)))
    zeros = jnp.zeros((NPAD, D), jnp.float32)
    il3 = jnp.stack([srcp.reshape(NT, EPT), dstp.reshape(NT, EPT)],
                    axis=-1).reshape(NT, 2 * NCH, K)
    eye2 = jnp.eye(2, D, dtype=jnp.float32)
    e01 = jnp.tile(eye2, (K // 2, 1))

    degp = _build_deg()(il3, e01, zeros)
    degq = degp[:, :, :2].transpose(1, 0, 2).reshape(NPAD, NC * 2)

    b = [b0.reshape(1, D), b1.reshape(1, D), b2.reshape(1, D)]
    g = [g0.reshape(1, D), g1.reshape(1, D), g2.reshape(1, D)]
    be = [be0.reshape(1, D), be1.reshape(1, D), be2.reshape(1, D)]

    hp = _tc_pre(x_pad, W0, degq)
    p0 = _build_agg()(hp, ilg, zeros)
    h0, hp = _tc_mid(p0, degq, b[0], g[0], be[0], W1)
    p1 = _build_agg()(hp, ilg, zeros)
    h1, hp = _tc_mid(p1, degq, b[1], g[1], be[1], W2)
    p2 = _build_agg()(hp, ilg, zeros)
    out = _tc_last(p2, degq, b[2], g[2], be[2], h0, h1,
                   lin1_W, lin1_b.reshape(1, D))
    return out
